# Initial kernel scaffold; baseline (speedup 1.0000x reference)
import functools

import jax
import jax.numpy as jnp
from jax.experimental import pallas as pl

_QB = 512


def _knn_kernel(klat_ref, klon_ref, qlat_ref, qlon_ref, xf_ref, mf_ref,
                interp_ref, dens_ref, *, n_keys):
    klat = klat_ref[0]
    klon = klon_ref[0]
    ckl = jnp.cos(klat)
    kx = ckl * jnp.cos(klon)
    ky = ckl * jnp.sin(klon)
    kz = jnp.sin(klat)
    qlat = qlat_ref[0]
    qlon = qlon_ref[0]
    cql = jnp.cos(qlat)
    qx = cql * jnp.cos(qlon)
    qy = cql * jnp.sin(qlon)
    qz = jnp.sin(qlat)

    s = kx * qx + ky * qy + kz * qz
    s = jnp.where(mf_ref[0] > 0, s, -2.0)

    iota = jax.lax.broadcasted_iota(jnp.int32, s.shape, 0)

    wts = []
    dists = []
    onehots = []
    for _ in range(3):
        m = jnp.max(s, axis=0, keepdims=True)
        idx = jnp.min(jnp.where(s == m, iota, n_keys), axis=0, keepdims=True)
        onehot = iota == idx
        s = jnp.where(onehot, -3.0, s)
        h = jnp.clip((1.0 - m) * 0.5, 0.0, 1.0)
        d = 2.0 * jnp.arcsin(jnp.sqrt(h))
        dists.append(d)
        wts.append(1.0 / (d + 1e-6))
        onehots.append(onehot)

    wsum = wts[0] + wts[1] + wts[2]
    wmat = (jnp.where(onehots[0], wts[0] / wsum, 0.0)
            + jnp.where(onehots[1], wts[1] / wsum, 0.0)
            + jnp.where(onehots[2], wts[2] / wsum, 0.0))

    interp = jax.lax.dot_general(
        wmat, xf_ref[0],
        dimension_numbers=(((0,), (0,)), ((), ())),
        preferred_element_type=jnp.float32,
        precision=jax.lax.Precision.HIGHEST)
    interp_ref[0] = interp

    dens = (jnp.exp(-dists[0]) + jnp.exp(-dists[1]) + jnp.exp(-dists[2])) * (1.0 / 3.0)
    dens_ref[0] = 1.0 - dens


def kernel(x, coords_input, coords_output, mask):
    b, nt, n, nv, c = x.shape
    B = b * nt
    N = n * nv
    xf = x.reshape(B, N, c)
    ci = coords_input.reshape(B, N, 2)
    co = coords_output.reshape(B, N, 2)
    klat = ci[..., 0:1]
    klon = ci[..., 1:2]
    qlat = co[..., 0]
    qlon = co[..., 1]
    mf = mask.reshape(B, N, 1).astype(jnp.float32)

    qb = _QB
    grid = (B, N // qb)
    interp, dens = pl.pallas_call(
        functools.partial(_knn_kernel, n_keys=N),
        grid=grid,
        in_specs=[
            pl.BlockSpec((1, N, 1), lambda i, j: (i, 0, 0)),
            pl.BlockSpec((1, N, 1), lambda i, j: (i, 0, 0)),
            pl.BlockSpec((1, qb), lambda i, j: (i, j)),
            pl.BlockSpec((1, qb), lambda i, j: (i, j)),
            pl.BlockSpec((1, N, c), lambda i, j: (i, 0, 0)),
            pl.BlockSpec((1, N, 1), lambda i, j: (i, 0, 0)),
        ],
        out_specs=[
            pl.BlockSpec((1, qb, c), lambda i, j: (i, j, 0)),
            pl.BlockSpec((1, qb), lambda i, j: (i, j)),
        ],
        out_shape=[
            jax.ShapeDtypeStruct((B, N, c), jnp.float32),
            jax.ShapeDtypeStruct((B, N), jnp.float32),
        ],
    )(klat, klon, qlat, qlon, xf, mf)

    out = interp.reshape(b, nt, N, c)
    density_emb = dens.reshape(b, nt, N)
    return out, density_emb


# speedup vs baseline: 14.7703x; 14.7703x over previous
import functools

import jax
import jax.numpy as jnp
from jax.experimental import pallas as pl

_QB = 512


def _knn_kernel(klat_ref, klon_ref, qlat_ref, qlon_ref, xf_ref, mf_ref,
                interp_ref, dens_ref, *, n_keys):
    klat = klat_ref[0]
    klon = klon_ref[0]
    ckl = jnp.cos(klat)
    kx = ckl * jnp.cos(klon)
    ky = ckl * jnp.sin(klon)
    kz = jnp.sin(klat)
    qlat = qlat_ref[0, 0]
    qlon = qlon_ref[0, 0]
    cql = jnp.cos(qlat)
    qx = cql * jnp.cos(qlon)
    qy = cql * jnp.sin(qlon)
    qz = jnp.sin(qlat)

    s = kx * qx + ky * qy + kz * qz
    s = jnp.where(mf_ref[0] > 0, s, -2.0)

    iota = jax.lax.broadcasted_iota(jnp.int32, s.shape, 0)

    wts = []
    dists = []
    onehots = []
    for _ in range(3):
        m = jnp.max(s, axis=0, keepdims=True)
        idx = jnp.min(jnp.where(s == m, iota, n_keys), axis=0, keepdims=True)
        onehot = iota == idx
        s = jnp.where(onehot, -3.0, s)
        h = jnp.clip((1.0 - m) * 0.5, 0.0, 1.0)
        d = 2.0 * jnp.arctan2(jnp.sqrt(h), jnp.sqrt(1.0 - h))
        dists.append(d)
        wts.append(1.0 / (d + 1e-6))
        onehots.append(onehot)

    wsum = wts[0] + wts[1] + wts[2]
    wmat = (jnp.where(onehots[0], wts[0] / wsum, 0.0)
            + jnp.where(onehots[1], wts[1] / wsum, 0.0)
            + jnp.where(onehots[2], wts[2] / wsum, 0.0))

    interp = jax.lax.dot_general(
        wmat, xf_ref[0],
        dimension_numbers=(((0,), (0,)), ((), ())),
        preferred_element_type=jnp.float32,
        precision=jax.lax.Precision.HIGHEST)
    interp_ref[0] = interp

    dens = (jnp.exp(-dists[0]) + jnp.exp(-dists[1]) + jnp.exp(-dists[2])) * (1.0 / 3.0)
    dens_ref[0, 0] = 1.0 - dens


def kernel(x, coords_input, coords_output, mask):
    b, nt, n, nv, c = x.shape
    B = b * nt
    N = n * nv
    xf = x.reshape(B, N, c)
    ci = coords_input.reshape(B, N, 2)
    co = coords_output.reshape(B, N, 2)
    qb = _QB
    nq = N // qb
    klat = ci[..., 0:1]
    klon = ci[..., 1:2]
    qlat = co[..., 0].reshape(B, nq, 1, qb)
    qlon = co[..., 1].reshape(B, nq, 1, qb)
    mf = mask.reshape(B, N, 1).astype(jnp.float32)

    grid = (B, nq)
    interp, dens = pl.pallas_call(
        functools.partial(_knn_kernel, n_keys=N),
        grid=grid,
        in_specs=[
            pl.BlockSpec((1, N, 1), lambda i, j: (i, 0, 0)),
            pl.BlockSpec((1, N, 1), lambda i, j: (i, 0, 0)),
            pl.BlockSpec((1, 1, 1, qb), lambda i, j: (i, j, 0, 0)),
            pl.BlockSpec((1, 1, 1, qb), lambda i, j: (i, j, 0, 0)),
            pl.BlockSpec((1, N, c), lambda i, j: (i, 0, 0)),
            pl.BlockSpec((1, N, 1), lambda i, j: (i, 0, 0)),
        ],
        out_specs=[
            pl.BlockSpec((1, qb, c), lambda i, j: (i, j, 0)),
            pl.BlockSpec((1, 1, 1, qb), lambda i, j: (i, j, 0, 0)),
        ],
        out_shape=[
            jax.ShapeDtypeStruct((B, N, c), jnp.float32),
            jax.ShapeDtypeStruct((B, nq, 1, qb), jnp.float32),
        ],
    )(klat, klon, qlat, qlon, xf, mf)

    out = interp.reshape(b, nt, N, c)
    density_emb = dens.reshape(b, nt, N)
    return out, density_emb


# speedup vs baseline: 23.2265x; 1.5725x over previous
import functools

import jax
import jax.numpy as jnp
from jax.experimental import pallas as pl

_QB = 512


def _knn_kernel(kx_ref, ky_ref, kz_ref, qx_ref, qy_ref, qz_ref, xf_ref, mf_ref,
                interp_ref, dens_ref, *, n_keys):
    kx = kx_ref[0]
    ky = ky_ref[0]
    kz = kz_ref[0]
    qx = qx_ref[0, 0]
    qy = qy_ref[0, 0]
    qz = qz_ref[0, 0]

    s = kx * qx + ky * qy + kz * qz
    s = jnp.where(mf_ref[0] > 0, s, -2.0)

    iota = jax.lax.broadcasted_iota(jnp.int32, s.shape, 0)

    wts = []
    dists = []
    onehots = []
    for _ in range(3):
        m = jnp.max(s, axis=0, keepdims=True)
        idx = jnp.min(jnp.where(s == m, iota, n_keys), axis=0, keepdims=True)
        onehot = iota == idx
        s = jnp.where(onehot, -3.0, s)
        h = jnp.clip((1.0 - m) * 0.5, 0.0, 1.0)
        d = 2.0 * jnp.arctan2(jnp.sqrt(h), jnp.sqrt(1.0 - h))
        dists.append(d)
        wts.append(1.0 / (d + 1e-6))
        onehots.append(onehot)

    wsum = wts[0] + wts[1] + wts[2]
    wmat = (jnp.where(onehots[0], wts[0] / wsum, 0.0)
            + jnp.where(onehots[1], wts[1] / wsum, 0.0)
            + jnp.where(onehots[2], wts[2] / wsum, 0.0))

    xf = xf_ref[0]
    w_hi = wmat.astype(jnp.bfloat16)
    w_lo = (wmat - w_hi.astype(jnp.float32)).astype(jnp.bfloat16)
    x_hi = xf.astype(jnp.bfloat16)
    x_lo = (xf - x_hi.astype(jnp.float32)).astype(jnp.bfloat16)
    dims = (((0,), (0,)), ((), ()))

    def mm(a, bmat):
        return jax.lax.dot_general(a, bmat, dimension_numbers=dims,
                                   preferred_element_type=jnp.float32)

    interp = mm(w_hi, x_hi) + (mm(w_hi, x_lo) + mm(w_lo, x_hi))
    interp_ref[0] = interp

    dens = (jnp.exp(-dists[0]) + jnp.exp(-dists[1]) + jnp.exp(-dists[2])) * (1.0 / 3.0)
    dens_ref[0, 0] = 1.0 - dens


def kernel(x, coords_input, coords_output, mask):
    b, nt, n, nv, c = x.shape
    B = b * nt
    N = n * nv
    xf = x.reshape(B, N, c)
    ci = coords_input.reshape(B, N, 2)
    co = coords_output.reshape(B, N, 2)
    qb = _QB
    nq = N // qb
    klat = ci[..., 0]
    klon = ci[..., 1]
    ckl = jnp.cos(klat)
    kx = (ckl * jnp.cos(klon))[..., None]
    ky = (ckl * jnp.sin(klon))[..., None]
    kz = jnp.sin(klat)[..., None]
    qlat = co[..., 0]
    qlon = co[..., 1]
    cql = jnp.cos(qlat)
    qx = (cql * jnp.cos(qlon)).reshape(B, nq, 1, qb)
    qy = (cql * jnp.sin(qlon)).reshape(B, nq, 1, qb)
    qz = jnp.sin(qlat).reshape(B, nq, 1, qb)
    mf = mask.reshape(B, N, 1).astype(jnp.float32)

    grid = (B, nq)
    interp, dens = pl.pallas_call(
        functools.partial(_knn_kernel, n_keys=N),
        grid=grid,
        in_specs=[
            pl.BlockSpec((1, N, 1), lambda i, j: (i, 0, 0)),
            pl.BlockSpec((1, N, 1), lambda i, j: (i, 0, 0)),
            pl.BlockSpec((1, N, 1), lambda i, j: (i, 0, 0)),
            pl.BlockSpec((1, 1, 1, qb), lambda i, j: (i, j, 0, 0)),
            pl.BlockSpec((1, 1, 1, qb), lambda i, j: (i, j, 0, 0)),
            pl.BlockSpec((1, 1, 1, qb), lambda i, j: (i, j, 0, 0)),
            pl.BlockSpec((1, N, c), lambda i, j: (i, 0, 0)),
            pl.BlockSpec((1, N, 1), lambda i, j: (i, 0, 0)),
        ],
        out_specs=[
            pl.BlockSpec((1, qb, c), lambda i, j: (i, j, 0)),
            pl.BlockSpec((1, 1, 1, qb), lambda i, j: (i, j, 0, 0)),
        ],
        out_shape=[
            jax.ShapeDtypeStruct((B, N, c), jnp.float32),
            jax.ShapeDtypeStruct((B, nq, 1, qb), jnp.float32),
        ],
    )(kx, ky, kz, qx, qy, qz, xf, mf)

    out = interp.reshape(b, nt, N, c)
    density_emb = dens.reshape(b, nt, N)
    return out, density_emb
